# speedup vs baseline: 1.5285x; 1.5285x over previous
import functools

import jax
import jax.numpy as jnp
from jax import lax
from jax.experimental import pallas as pl
from jax.experimental.pallas import tpu as pltpu
from jax.experimental.pallas import tpu_sc as plsc

BATCH_N = 16384
DIM = 128

_info = plsc.get_sparse_core_info()
_NC = _info.num_cores
_NS = _info.num_subcores
_NW = _NC * _NS
_BPW = BATCH_N // _NW
_CHUNK = 128
_NCH = _BPW // _CHUNK

_mesh = plsc.VectorSubcoreMesh(core_axis_name="c", subcore_axis_name="s")


@functools.partial(
    pl.kernel,
    out_type=(
        jax.ShapeDtypeStruct((BATCH_N, DIM), jnp.float32),
        jax.ShapeDtypeStruct((BATCH_N, DIM), jnp.float32),
    ),
    mesh=_mesh,
    scratch_types=[
        pltpu.VMEM((_NCH, _CHUNK), jnp.int32),
        pltpu.VMEM((_CHUNK, DIM), jnp.float32),
        pltpu.VMEM((_CHUNK, DIM), jnp.float32),
        pltpu.SemaphoreType.DMA,
        pltpu.SemaphoreType.DMA,
    ],
)
def _dual_gather(ids_hbm, shape_hbm, tex_hbm, out_s, out_t,
                 idx_v, buf0, buf1, sem0, sem1):
    wid = lax.axis_index("s") * _NC + lax.axis_index("c")
    base = wid * _BPW
    pltpu.sync_copy(ids_hbm.at[pl.ds(wid * _NCH, _NCH)], idx_v)

    bufs = (buf0, buf1)
    sems = (sem0, sem1)
    tabs = (shape_hbm, tex_hbm)
    outs = (out_s, out_t)
    jobs = [(t, j) for t in range(2) for j in range(_NCH)]

    handles = [None, None]
    t0, j0 = jobs[0]
    handles[0] = pltpu.async_copy(tabs[t0].at[idx_v.at[j0]], bufs[0], sems[0])
    for i, (t, j) in enumerate(jobs):
        cur = i % 2
        if i + 1 < len(jobs):
            tn, jn = jobs[i + 1]
            handles[1 - cur] = pltpu.async_copy(
                tabs[tn].at[idx_v.at[jn]], bufs[1 - cur], sems[1 - cur])
        handles[cur].wait()
        pltpu.sync_copy(bufs[cur],
                        outs[t].at[pl.ds(base + j * _CHUNK, _CHUNK)])


def kernel(object_ids, shape_code, texture_code):
    ids2d = object_ids.astype(jnp.int32).reshape(_NW * _NCH, _CHUNK)
    return _dual_gather(ids2d, shape_code, texture_code)


# speedup vs baseline: 1.5799x; 1.0336x over previous
import functools

import jax
import jax.numpy as jnp
from jax import lax
from jax.experimental import pallas as pl
from jax.experimental.pallas import tpu as pltpu
from jax.experimental.pallas import tpu_sc as plsc

BATCH_N = 16384
DIM = 128

_info = plsc.get_sparse_core_info()
_NC = _info.num_cores
_NS = _info.num_subcores
_NW = _NC * _NS
_BPW = BATCH_N // _NW
_CHUNK = 128
_NCH = _BPW // _CHUNK

_mesh = plsc.VectorSubcoreMesh(core_axis_name="c", subcore_axis_name="s")


@functools.partial(
    pl.kernel,
    out_type=(
        jax.ShapeDtypeStruct((BATCH_N, DIM), jnp.float32),
        jax.ShapeDtypeStruct((BATCH_N, DIM), jnp.float32),
    ),
    mesh=_mesh,
    scratch_types=[
        pltpu.VMEM((_NCH, _CHUNK), jnp.int32),
        pltpu.VMEM((4, _CHUNK, DIM), jnp.float32),
        pltpu.SemaphoreType.DMA,
        pltpu.SemaphoreType.DMA,
        pltpu.SemaphoreType.DMA,
        pltpu.SemaphoreType.DMA,
    ],
)
def _dual_gather(ids_hbm, shape_hbm, tex_hbm, out_s, out_t,
                 idx_v, bufs_v, sem0, sem1, sem2, sem3):
    _NBUF = 4
    wid = lax.axis_index("s") * _NC + lax.axis_index("c")
    base = wid * _BPW
    pltpu.sync_copy(ids_hbm.at[pl.ds(wid * _NCH, _NCH)], idx_v)

    sems = (sem0, sem1, sem2, sem3)
    tabs = (shape_hbm, tex_hbm)
    outs = (out_s, out_t)
    jobs = [(t, j) for t in range(2) for j in range(_NCH)]

    handles = [None] * _NBUF
    for i in range(_NBUF - 1):
        t, j = jobs[i]
        handles[i] = pltpu.async_copy(
            tabs[t].at[idx_v.at[j]], bufs_v.at[i], sems[i])
    for i, (t, j) in enumerate(jobs):
        cur = i % _NBUF
        nx = i + _NBUF - 1
        if nx < len(jobs):
            tn, jn = jobs[nx]
            b = nx % _NBUF
            handles[b] = pltpu.async_copy(
                tabs[tn].at[idx_v.at[jn]], bufs_v.at[b], sems[b])
        handles[cur].wait()
        pltpu.sync_copy(bufs_v.at[cur],
                        outs[t].at[pl.ds(base + j * _CHUNK, _CHUNK)])


def kernel(object_ids, shape_code, texture_code):
    ids2d = object_ids.astype(jnp.int32).reshape(_NW * _NCH, _CHUNK)
    return _dual_gather(ids2d, shape_code, texture_code)
